# Initial kernel scaffold; baseline (speedup 1.0000x reference)
import jax
import jax.numpy as jnp
from jax.experimental import pallas as pl


def kernel(x, Wr, W1, b1, W2, b2):
    raise NotImplementedError("write your pallas kernel here")


# speedup vs baseline: 1.8944x; 1.8944x over previous
import jax
import jax.numpy as jnp
from jax.experimental import pallas as pl
from jax.experimental.pallas import tpu as pltpu

_E = 16
_S = 4
_D = 768
_HS = 768
_ES = 192
_T = 4096
_CAP = int(1.5 * _T / _E)
_TT = 512

_I32_MIN = jnp.iinfo(jnp.int32).min
_I32_MAX = jnp.iinfo(jnp.int32).max


def _routing_weights(logits):
    bits = jax.lax.bitcast_convert_type(logits, jnp.int32)
    key = jnp.where(bits >= 0, bits, bits ^ jnp.int32(0x7FFFFFFF))

    lo0 = jnp.full((_E, 1), _I32_MIN, jnp.int32)
    hi0 = jnp.full((_E, 1), _I32_MAX, jnp.int32)

    def srch(_, lh):
        lo, hi = lh
        mid = (lo >> 1) + (hi >> 1) + (lo & hi & 1)
        c = jnp.sum((key > mid).astype(jnp.int32), axis=1, keepdims=True)
        big = c >= _CAP
        return jnp.where(big, mid + 1, lo), jnp.where(big, hi, mid)

    thr, _ = jax.lax.fori_loop(0, 32, srch, (lo0, hi0))

    gt = key > thr
    n_gt = jnp.sum(gt.astype(jnp.int32), axis=1, keepdims=True)
    need = _CAP - n_gt
    eqm = key == thr
    tidx = jax.lax.broadcasted_iota(jnp.int32, (_E, _T), 1)

    def srch2(_, lh):
        lo, hi = lh
        mid = (lo + hi) >> 1
        cnt = jnp.sum((eqm & (tidx <= mid)).astype(jnp.int32), axis=1,
                      keepdims=True)
        ok = cnt >= need
        return jnp.where(ok, lo, mid + 1), jnp.where(ok, mid, hi)

    cidx, _ = jax.lax.fori_loop(
        0, 12, srch2,
        (jnp.zeros((_E, 1), jnp.int32), jnp.full((_E, 1), _T - 1, jnp.int32)))

    sel = gt | (eqm & (tidx <= cidx) & (need > 0))
    m = jnp.max(logits, axis=1, keepdims=True)
    ex = jnp.where(sel, jnp.exp(logits - m), 0.0)
    z = jnp.sum(ex, axis=1, keepdims=True)
    return ex / z


def _gelu(h):
    return 0.5 * h * (1.0 + jax.lax.erf(h * 0.7071067811865476))


def _moe_kernel(x_ref, wr_ref, w1_ref, b1_ref, w2_ref, b2_ref,
                out_ref, wt_out_ref, wt_ref):
    t = pl.program_id(0)
    e = pl.program_id(1)

    @pl.when((t == 0) & (e == 0))
    def _():
        logits = jax.lax.dot_general(
            wr_ref[...], x_ref[...], (((1,), (1,)), ((), ())),
            preferred_element_type=jnp.float32)
        wt_ref[...] = _routing_weights(logits)

    @pl.when((t == pl.num_programs(0) - 1) & (e == pl.num_programs(1) - 1))
    def _():
        wt_out_ref[...] = wt_ref[...]

    xt = x_ref[pl.ds(t * _TT, _TT), :].astype(jnp.bfloat16)
    outs = []
    for s in range(_S):
        w1s = w1_ref[0, s].astype(jnp.bfloat16)
        h = jax.lax.dot_general(xt, w1s, (((1,), (0,)), ((), ())),
                                preferred_element_type=jnp.float32)
        h = _gelu(h + b1_ref[0, s][None, :])
        w2s = w2_ref[0, s].astype(jnp.bfloat16)
        o = jax.lax.dot_general(h.astype(jnp.bfloat16), w2s,
                                (((1,), (0,)), ((), ())),
                                preferred_element_type=jnp.float32)
        outs.append(o + b2_ref[0, s][None, :])
    o_full = jnp.concatenate(outs, axis=1)
    wt_blk = wt_ref[:, pl.ds(t * _TT, _TT)]
    onehot = (jax.lax.broadcasted_iota(jnp.int32, (_E, 1), 0) == e
              ).astype(jnp.float32)
    wcol = jax.lax.dot_general(wt_blk, onehot, (((0,), (0,)), ((), ())),
                               preferred_element_type=jnp.float32)
    contrib = o_full * wcol

    @pl.when(e == 0)
    def _():
        out_ref[...] = contrib

    @pl.when(e != 0)
    def _():
        out_ref[...] += contrib


def kernel(x, Wr, W1, b1, W2, b2):
    orig_shape = x.shape
    xf = x.reshape(-1, x.shape[-1])
    out, _ = pl.pallas_call(
        _moe_kernel,
        grid=(_T // _TT, _E),
        in_specs=[
            pl.BlockSpec((_T, _D), lambda t, e: (0, 0)),
            pl.BlockSpec((_E, _D), lambda t, e: (0, 0)),
            pl.BlockSpec((1, _S, _D, _HS), lambda t, e: (e, 0, 0, 0)),
            pl.BlockSpec((1, _S, _HS), lambda t, e: (e, 0, 0)),
            pl.BlockSpec((1, _S, _HS, _ES), lambda t, e: (e, 0, 0, 0)),
            pl.BlockSpec((1, _S, _ES), lambda t, e: (e, 0, 0)),
        ],
        out_specs=[
            pl.BlockSpec((_TT, _D), lambda t, e: (t, 0)),
            pl.BlockSpec((_E, _T), lambda t, e: (0, 0)),
        ],
        out_shape=[
            jax.ShapeDtypeStruct((_T, _D), jnp.float32),
            jax.ShapeDtypeStruct((_E, _T), jnp.float32),
        ],
        scratch_shapes=[pltpu.VMEM((_E, _T), jnp.float32)],
        compiler_params=pltpu.CompilerParams(
            dimension_semantics=("arbitrary", "arbitrary")),
    )(xf, Wr, W1, b1, W2, b2)
    return out.reshape(orig_shape)


# speedup vs baseline: 5.1932x; 2.7413x over previous
import functools

import jax
import jax.numpy as jnp
from jax import lax
from jax.experimental import pallas as pl
from jax.experimental.pallas import tpu as pltpu
import jax.experimental.pallas.tpu_sc as plsc

_E = 16
_S = 4
_D = 768
_HS = 768
_ES = 192
_T = 4096
_CAP = int(1.5 * _T / _E)

_I32_MIN = jnp.iinfo(jnp.int32).min
_I32_MAX = jnp.iinfo(jnp.int32).max


def _keys(logits):
    bits = lax.bitcast_convert_type(logits, jnp.int32)
    return jnp.where(bits >= 0, bits, bits ^ jnp.int32(0x7FFFFFFF))


def _search(key, axis, shape1):
    lo0 = jnp.full(shape1, _I32_MIN, jnp.int32)
    hi0 = jnp.full(shape1, _I32_MAX, jnp.int32)

    def srch(_, lh):
        lo, hi = lh
        mid = (lo >> 1) + (hi >> 1) + (lo & hi & 1)
        c = jnp.sum((key > mid).astype(jnp.int32), axis=axis, keepdims=True)
        big = c >= _CAP
        return jnp.where(big, mid + 1, lo), jnp.where(big, hi, mid)

    thr, _ = lax.fori_loop(0, 32, srch, (lo0, hi0))
    gt = key > thr
    n_gt = jnp.sum(gt.astype(jnp.int32), axis=axis, keepdims=True)
    need = _CAP - n_gt
    eqm = key == thr
    tidx = lax.broadcasted_iota(jnp.int32, key.shape, axis)

    def srch2(_, lh):
        lo, hi = lh
        mid = (lo + hi) >> 1
        cnt = jnp.sum((eqm & (tidx <= mid)).astype(jnp.int32), axis=axis,
                      keepdims=True)
        ok = cnt >= need
        return jnp.where(ok, lo, mid + 1), jnp.where(ok, mid, hi)

    cidx, _ = lax.fori_loop(
        0, 12, srch2,
        (jnp.zeros(shape1, jnp.int32), jnp.full(shape1, _T - 1, jnp.int32)))
    sel = gt | (eqm & (tidx <= cidx) & (need > 0))
    return sel


def _prefix_incl(v, axis):
    n = v.shape[axis]
    idx = lax.broadcasted_iota(jnp.int32, v.shape, axis)
    sh = 1
    while sh < n:
        r = pltpu.roll(v, sh, axis)
        v = v + jnp.where(idx >= sh, r, 0)
        sh *= 2
    return v


def _routing_kernel(x_ref, wr_ref, wt_ref, pos_ref, posm_ref):
    lg1 = lax.dot_general(wr_ref[...], x_ref[...], (((1,), (1,)), ((), ())),
                          preferred_element_type=jnp.float32)
    key1 = _keys(lg1)
    sel1 = _search(key1, 1, (_E, 1))
    si1 = jnp.where(sel1, 1, 0)
    pref1 = _prefix_incl(si1, 1)
    pos_ref[...] = jnp.where(sel1, pref1 - 1, 0)
    m = jnp.max(lg1, axis=1, keepdims=True)
    ex = jnp.where(sel1, jnp.exp(lg1 - m), 0.0)
    z = jnp.sum(ex, axis=1, keepdims=True)
    wt_ref[...] = ex / z
    lg2 = lax.dot_general(x_ref[...], wr_ref[...], (((1,), (1,)), ((), ())),
                          preferred_element_type=jnp.float32)
    key2 = _keys(lg2)
    sel2 = _search(key2, 0, (1, _E))
    si2 = jnp.where(sel2, 1, 0)
    pref2 = _prefix_incl(si2, 0)
    posm_ref[...] = jnp.where(sel2, pref2 - 1, _CAP)


def _routing(xf, Wr):
    return pl.pallas_call(
        _routing_kernel,
        out_shape=[
            jax.ShapeDtypeStruct((_E, _T), jnp.float32),
            jax.ShapeDtypeStruct((_E, _T), jnp.int32),
            jax.ShapeDtypeStruct((_T, _E), jnp.int32),
        ],
    )(xf, Wr)


_mesh = plsc.VectorSubcoreMesh(core_axis_name="c", subcore_axis_name="s",
                               num_cores=2, num_subcores=16)
_GCH = 48
_GH = _CAP // 2


@functools.partial(
    pl.kernel, mesh=_mesh,
    out_type=[jax.ShapeDtypeStruct((_E, _CAP, _D), jnp.float32),
              jax.ShapeDtypeStruct((_E, _CAP), jnp.float32)],
    scratch_types=[pltpu.VMEM((_T,), jnp.float32),
                   pltpu.VMEM((_T,), jnp.int32),
                   pltpu.VMEM((_CAP,), jnp.int32),
                   pltpu.VMEM((_CAP,), jnp.float32),
                   pltpu.VMEM((_GCH, _D), jnp.float32),
                   pltpu.SemaphoreType.DMA],
    compiler_params=pltpu.CompilerParams(needs_layout_passes=False),
)
def _sc_gather(wt_hbm, pos_hbm, xf_hbm, g_hbm, wv_hbm,
               row_v, pos_v, idx_v, wv_v, rows_v, sem):
    c = lax.axis_index("c")
    e = lax.axis_index("s")
    pltpu.sync_copy(wt_hbm.at[e], row_v)
    pltpu.sync_copy(pos_hbm.at[e], pos_v)
    z16i = jnp.zeros((16,), jnp.int32)
    z16f = jnp.zeros((16,), jnp.float32)

    def zinit(i, carry):
        idx_v[pl.ds(i * 16, 16)] = z16i
        wv_v[pl.ds(i * 16, 16)] = z16f
        return carry

    lax.fori_loop(0, _CAP // 16, zinit, 0)

    def body(ci, carry):
        v = row_v[pl.ds(ci * 16, 16)]
        p = pos_v[pl.ds(ci * 16, 16)]
        m = v > 0.0
        tok = ci * 16 + lax.iota(jnp.int32, 16)
        plsc.store_scatter(idx_v, [p], tok, mask=m)
        plsc.store_scatter(wv_v, [p], v, mask=m)
        return carry

    lax.fori_loop(0, _T // 16, body, 0)

    @pl.when(c == 0)
    def _():
        pltpu.sync_copy(wv_v, wv_hbm.at[e])

    for j in range(_GH // _GCH):
        off = pl.multiple_of(c * _GH + j * _GCH, 8)
        pltpu.async_copy(xf_hbm.at[idx_v.at[pl.ds(off, _GCH)]],
                         rows_v, sem).wait()
        pltpu.sync_copy(rows_v, g_hbm.at[e, pl.ds(off, _GCH)])


def _gelu(h):
    return 0.5 * h * (1.0 + lax.erf(h * 0.7071067811865476))


def _mlp_kernel(g_ref, wv_ref, posm_ref, w1_ref, b1_ref, w2_ref, b2_ref,
                out_ref):
    e = pl.program_id(0)

    @pl.when(e == 0)
    def _():
        out_ref[...] = jnp.zeros_like(out_ref)

    xg = g_ref[0].astype(jnp.bfloat16)
    outs = []
    for s in range(_S):
        w1s = w1_ref[0, s].astype(jnp.bfloat16)
        h = lax.dot_general(xg, w1s, (((1,), (0,)), ((), ())),
                            preferred_element_type=jnp.float32)
        h = _gelu(h + b1_ref[0, s][None, :])
        w2s = w2_ref[0, s].astype(jnp.bfloat16)
        o = lax.dot_general(h.astype(jnp.bfloat16), w2s,
                            (((1,), (0,)), ((), ())),
                            preferred_element_type=jnp.float32)
        outs.append(o + b2_ref[0, s][None, :])
    eo = jnp.concatenate(outs, axis=1) * wv_ref[0]
    lane_e = lax.broadcasted_iota(jnp.int32, (1, _E), 1) == e
    pcol = jnp.sum(jnp.where(lane_e, posm_ref[...], 0), axis=1,
                   keepdims=True)
    a = (pcol == lax.broadcasted_iota(jnp.int32, (1, _CAP), 1)
         ).astype(jnp.bfloat16)
    contrib = lax.dot_general(a, eo.astype(jnp.bfloat16),
                              (((1,), (0,)), ((), ())),
                              preferred_element_type=jnp.float32)
    out_ref[...] += contrib


def _mlp(g, wv3, posm, W1, b1, W2, b2):
    return pl.pallas_call(
        _mlp_kernel,
        grid=(_E,),
        in_specs=[
            pl.BlockSpec((1, _CAP, _D), lambda e: (e, 0, 0)),
            pl.BlockSpec((1, _CAP, 1), lambda e: (e, 0, 0)),
            pl.BlockSpec((_T, _E), lambda e: (0, 0)),
            pl.BlockSpec((1, _S, _D, _HS), lambda e: (e, 0, 0, 0)),
            pl.BlockSpec((1, _S, _HS), lambda e: (e, 0, 0)),
            pl.BlockSpec((1, _S, _HS, _ES), lambda e: (e, 0, 0, 0)),
            pl.BlockSpec((1, _S, _ES), lambda e: (e, 0, 0)),
        ],
        out_specs=pl.BlockSpec((_T, _D), lambda e: (0, 0)),
        out_shape=jax.ShapeDtypeStruct((_T, _D), jnp.float32),
        compiler_params=pltpu.CompilerParams(
            dimension_semantics=("arbitrary",)),
    )(g, wv3, posm, W1, b1, W2, b2)


def kernel(x, Wr, W1, b1, W2, b2):
    orig_shape = x.shape
    xf = x.reshape(-1, x.shape[-1])
    wt, pos, posm = _routing(xf, Wr)
    g, wv = _sc_gather(wt, pos, xf)
    out = _mlp(g, wv.reshape(_E, _CAP, 1), posm, W1, b1, W2, b2)
    return out.reshape(orig_shape)
